# baseline (speedup 1.0000x reference)
import jax
import jax.numpy as jnp
from jax.experimental import pallas as pl
from jax.experimental.pallas import tpu as pltpu

QUEUE = 65536
DIM = 128
BATCH = 4096
NCH = QUEUE // BATCH
LROWS = BATCH // 128


def _enqueue_kernel(s_ref, qe_ref, ql_ref, emb_ref, lab_ref,
                    oe_ref, ol_ref, z_ref, sem):
    wk = s_ref[0]

    for k in range(NCH):
        @pl.when(wk != k)
        def _(k=k):
            pltpu.make_async_copy(
                qe_ref.at[pl.ds(k * BATCH, BATCH), :],
                oe_ref.at[pl.ds(k * BATCH, BATCH), :],
                sem.at[k],
            ).start()
            pltpu.make_async_copy(
                ql_ref.at[pl.ds(k * LROWS, LROWS), :],
                ol_ref.at[pl.ds(k * LROWS, LROWS), :],
                sem.at[NCH + k],
            ).start()

    x = emb_ref[...]
    n = jnp.sqrt(jnp.sum(x * x, axis=1, keepdims=True))
    z_ref[...] = x / jnp.maximum(n, 1e-12)

    zcopy = pltpu.make_async_copy(
        z_ref, oe_ref.at[pl.ds(wk * BATCH, BATCH), :], sem.at[2 * NCH])
    lcopy = pltpu.make_async_copy(
        lab_ref, ol_ref.at[pl.ds(wk * LROWS, LROWS), :], sem.at[2 * NCH + 1])
    zcopy.start()
    lcopy.start()

    for k in range(NCH):
        @pl.when(wk != k)
        def _(k=k):
            pltpu.make_async_copy(
                qe_ref.at[pl.ds(k * BATCH, BATCH), :],
                oe_ref.at[pl.ds(k * BATCH, BATCH), :],
                sem.at[k],
            ).wait()
            pltpu.make_async_copy(
                ql_ref.at[pl.ds(k * LROWS, LROWS), :],
                ol_ref.at[pl.ds(k * LROWS, LROWS), :],
                sem.at[NCH + k],
            ).wait()

    zcopy.wait()
    lcopy.wait()


def kernel(embeddings, labels, queue_embeds, queue_labels, queue_ptr):
    ldtype = queue_labels.dtype
    ql2 = queue_labels.reshape(QUEUE // 128, 128)
    lab2 = labels.astype(ldtype).reshape(LROWS, 128)
    s_blk = jnp.reshape(
        jax.lax.rem(queue_ptr.astype(jnp.int32) // BATCH, NCH), (1,)
    )

    grid_spec = pltpu.PrefetchScalarGridSpec(
        num_scalar_prefetch=1,
        grid=(1,),
        in_specs=[
            pl.BlockSpec(memory_space=pl.ANY),
            pl.BlockSpec(memory_space=pl.ANY),
            pl.BlockSpec((BATCH, DIM), lambda i, s: (0, 0)),
            pl.BlockSpec((LROWS, 128), lambda i, s: (0, 0)),
        ],
        out_specs=[
            pl.BlockSpec(memory_space=pl.ANY),
            pl.BlockSpec(memory_space=pl.ANY),
        ],
        scratch_shapes=[
            pltpu.VMEM((BATCH, DIM), jnp.float32),
            pltpu.SemaphoreType.DMA((2 * NCH + 2,)),
        ],
    )

    new_qe, new_ql2 = pl.pallas_call(
        _enqueue_kernel,
        grid_spec=grid_spec,
        out_shape=[
            jax.ShapeDtypeStruct((QUEUE, DIM), queue_embeds.dtype),
            jax.ShapeDtypeStruct((QUEUE // 128, 128), ldtype),
        ],
    )(s_blk, queue_embeds, ql2, embeddings, lab2)

    new_ptr = ((queue_ptr + BATCH) % QUEUE).astype(queue_ptr.dtype)
    return (new_qe, new_ql2.reshape(QUEUE), new_ptr)


# speedup vs baseline: 33.9287x; 33.9287x over previous
import jax
import jax.numpy as jnp
from jax.experimental import pallas as pl
from jax.experimental.pallas import tpu as pltpu

QUEUE = 65536
DIM = 128
BATCH = 4096
BS = 4096
NB = QUEUE // BS
NW = BATCH // BS
LS = BS // 128


def _enqueue_kernel(s_ref, qe_ref, ql_ref, emb_ref, lab_ref, oe_ref, ol_ref):
    k = pl.program_id(0)
    off = jax.lax.rem(k - s_ref[0] + NB, NB)
    is_write = off < NW

    @pl.when(is_write)
    def _():
        x = emb_ref[...]
        n = jnp.sqrt(jnp.sum(x * x, axis=1, keepdims=True))
        oe_ref[...] = x / jnp.maximum(n, 1e-12)
        ol_ref[...] = lab_ref[...]

    @pl.when(jnp.logical_not(is_write))
    def _():
        oe_ref[...] = qe_ref[...]
        ol_ref[...] = ql_ref[...]


def _emb_block(k, s):
    off = jax.lax.rem(k - s[0] + NB, NB)
    return (jnp.where(off < NW, off, 0), 0)


def _queue_block(k, s):
    off = jax.lax.rem(k - s[0] + NB, NB)
    return (jnp.where(off < NW, s[0], k), 0)


def kernel(embeddings, labels, queue_embeds, queue_labels, queue_ptr):
    ldtype = queue_labels.dtype
    ql2 = queue_labels.reshape(QUEUE // 128, 128)
    lab2 = labels.astype(ldtype).reshape(BATCH // 128, 128)
    s_blk = jnp.reshape(
        jax.lax.rem(queue_ptr.astype(jnp.int32) // BS, NB), (1,)
    )

    grid_spec = pltpu.PrefetchScalarGridSpec(
        num_scalar_prefetch=1,
        grid=(NB,),
        in_specs=[
            pl.BlockSpec((BS, DIM), _queue_block),
            pl.BlockSpec((LS, 128), lambda k, s: (_queue_block(k, s)[0], 0)),
            pl.BlockSpec((BS, DIM), _emb_block),
            pl.BlockSpec((LS, 128), lambda k, s: (_emb_block(k, s)[0], 0)),
        ],
        out_specs=[
            pl.BlockSpec((BS, DIM), lambda k, s: (k, 0)),
            pl.BlockSpec((LS, 128), lambda k, s: (k, 0)),
        ],
    )

    new_qe, new_ql2 = pl.pallas_call(
        _enqueue_kernel,
        grid_spec=grid_spec,
        out_shape=[
            jax.ShapeDtypeStruct((QUEUE, DIM), queue_embeds.dtype),
            jax.ShapeDtypeStruct((QUEUE // 128, 128), ldtype),
        ],
    )(s_blk, queue_embeds, ql2, embeddings, lab2)

    new_ptr = ((queue_ptr + BATCH) % QUEUE).astype(queue_ptr.dtype)
    return (new_qe, new_ql2.reshape(QUEUE), new_ptr)
